# baseline (speedup 1.0000x reference)
import functools
import math

import jax
import jax.numpy as jnp
from jax import lax
from jax.experimental import pallas as pl
from jax.experimental.pallas import tpu as pltpu
from jax.experimental.pallas import tpu_sc as plsc

EMB = 128
SCALE = math.sqrt(EMB)

NC = 2
NS = 16
NW = NC * NS

CHUNK = 128
NBUF = 4


def _make_gather(n_tokens):
    per_w = n_tokens // NW
    n_stages = per_w // CHUNK
    mesh = plsc.VectorSubcoreMesh(core_axis_name="c", subcore_axis_name="s")

    @functools.partial(
        pl.kernel,
        mesh=mesh,
        out_type=jax.ShapeDtypeStruct((n_tokens, EMB), jnp.float32),
        scratch_types=[
            pltpu.VMEM((n_stages, CHUNK), jnp.int32),
        ] + [pltpu.VMEM((CHUNK, EMB), jnp.float32)] * NBUF
          + [pltpu.SemaphoreType.DMA] * (2 * NBUF),
    )
    def gather_kernel(tok_hbm, table_hbm, out_hbm, idx_v, *bufs):
        rows = bufs[:NBUF]
        gsem = bufs[NBUF:2 * NBUF]
        wsem = bufs[2 * NBUF:]
        wid = lax.axis_index("s") * NC + lax.axis_index("c")
        row_base = wid * per_w

        pltpu.sync_copy(tok_hbm.at[pl.ds(wid * n_stages, n_stages)], idx_v)

        def g_start(s, b):
            pltpu.async_copy(table_hbm.at[idx_v.at[s]], rows[b], gsem[b])

        def g_wait(b):
            pltpu.make_async_copy(table_hbm.at[idx_v.at[0]], rows[b],
                                  gsem[b]).wait()

        def w_start(s, b):
            pltpu.async_copy(rows[b],
                             out_hbm.at[pl.ds(row_base + s * CHUNK, CHUNK)],
                             wsem[b])

        def w_wait(b):
            pltpu.make_async_copy(rows[b], out_hbm.at[pl.ds(row_base, CHUNK)],
                                  wsem[b]).wait()

        for s0 in range(NBUF - 1):
            g_start(s0, s0)

        def scale_rows(b):
            buf = rows[b]

            def srow(r, carry):
                for rr in range(2):
                    for k in range(8):
                        sl = (r * 2 + rr, pl.ds(k * 16, 16))
                        buf[sl] = buf[sl] * SCALE
                return carry

            lax.fori_loop(0, CHUNK // 2, srow, 0)

        def phase(s, b):
            g_wait(b)
            nxt = (b + NBUF - 1) % NBUF

            @pl.when(s + NBUF - 1 < n_stages)
            def _():
                @pl.when(s >= 1)
                def _():
                    w_wait(nxt)
                g_start(s + NBUF - 1, nxt)

            scale_rows(b)
            w_start(s, b)

        def body(i, carry):
            s = i * NBUF
            for b in range(NBUF):
                phase(s + b, b)
            return carry

        lax.fori_loop(0, n_stages // NBUF, body, 0)
        for b in range(NBUF):
            w_wait(b)

    return gather_kernel


def kernel(tokens, embeddings):
    b, t = tokens.shape
    flat = tokens.reshape(b * t // CHUNK, CHUNK).astype(jnp.int32)
    out = _make_gather(b * t)(flat, embeddings)
    return out.reshape(b, t, EMB)
